# Initial kernel scaffold; baseline (speedup 1.0000x reference)
import jax
import jax.numpy as jnp
from jax.experimental import pallas as pl


def kernel(batch_x, edge_index, edge_vals, weight, bias):
    raise NotImplementedError("write your pallas kernel here")


# speedup vs baseline: 3.6169x; 3.6169x over previous
import functools

import jax
import jax.numpy as jnp
from jax import lax
from jax.experimental import pallas as pl
from jax.experimental.pallas import tpu as pltpu
from jax.experimental.pallas import tpu_sc as plsc

N_NODES = 10000
N_FEAT = 128
N_EDGES = 320000

NC = 2
NS = 16
LANES = 16

CHUNK = 64
CHUNKS_PER_TILE = 160
EDGES_PER_TILE = CHUNKS_PER_TILE * CHUNK
E_PAD = NC * NS * EDGES_PER_TILE

N_PAD = 10240
ROWS_PER_TILE = N_PAD // NS
ZCHUNK = 64

NBUF = 4
NIDX = 8


def _sc_segment_sum(table, idx3):
    mesh = plsc.VectorSubcoreMesh(
        core_axis_name="c", subcore_axis_name="s", num_cores=NC, num_subcores=NS
    )

    @functools.partial(
        pl.kernel,
        out_type=jax.ShapeDtypeStruct((NC, N_PAD, N_FEAT), jnp.float32),
        mesh=mesh,
        compiler_params=pltpu.CompilerParams(needs_layout_passes=False),
        scratch_types=(
            [pltpu.VMEM_SHARED((N_PAD, N_FEAT), jnp.float32)]
            + [pltpu.VMEM((CHUNK, N_FEAT), jnp.float32)] * NBUF
            + [pltpu.VMEM((3, CHUNK), jnp.int32)] * NIDX
            + [pltpu.SemaphoreType.DMA] * (NBUF + NBUF + NIDX)
        ),
    )
    def k(table_hbm, idx_hbm, out_hbm, acc_sp, *bufs):
        rows_bufs = bufs[:NBUF]
        ibufs = bufs[NBUF:NBUF + NIDX]
        gsems = bufs[NBUF + NIDX:2 * NBUF + NIDX]
        ssems = bufs[2 * NBUF + NIDX:3 * NBUF + NIDX]
        isems = bufs[3 * NBUF + NIDX:]
        c = lax.axis_index("c")
        s = lax.axis_index("s")
        wid = c * NS + s

        def fire_idx(kk, sl):
            pltpu.async_copy(idx_hbm.at[wid, kk], ibufs[sl], isems[sl])

        def wait_idx(kk, sl):
            pltpu.make_async_copy(idx_hbm.at[wid, kk], ibufs[sl], isems[sl]).wait()

        def fire_gather(kk, b, sl):
            pltpu.async_copy(table_hbm.at[ibufs[sl].at[0]], rows_bufs[b], gsems[b])

        def wait_gather(kk, b, sl):
            pltpu.make_async_copy(
                table_hbm.at[ibufs[sl].at[0]], rows_bufs[b], gsems[b]).wait()

        def fire_scatter(kk, b, sl):
            pltpu.async_copy(rows_bufs[b], acc_sp.at[ibufs[sl].at[1]],
                             ssems[b], add=True)

        def wait_scatter(kk, b, sl):
            pltpu.make_async_copy(rows_bufs[b], acc_sp.at[ibufs[sl].at[1]],
                                  ssems[b]).wait()

        def scale(kk, b, sl):
            rows = rows_bufs[b]
            ibuf = ibufs[sl]

            @plsc.parallel_loop(0, CHUNK, unroll=4)
            def _(e):
                vv = plsc.bitcast(
                    plsc.load_gather(
                        ibuf,
                        [jnp.full((LANES,), 2, jnp.int32),
                         jnp.full((LANES,), e, jnp.int32)],
                    ),
                    jnp.float32,
                )
                for q in range(N_FEAT // LANES):
                    rows[e, pl.ds(q * LANES, LANES)] = (
                        rows[e, pl.ds(q * LANES, LANES)] * vv)

        def step(kk, b, sl, do_wait_scatter, do_fire_gather, do_fire_idx):
            if do_wait_scatter:
                wait_scatter(kk - 2, (b + 2) % NBUF, (sl + 6) % NIDX)
            if do_fire_gather:
                wait_idx(kk + 2, (sl + 2) % NIDX)
                fire_gather(kk + 2, (b + 2) % NBUF, (sl + 2) % NIDX)
            if do_fire_idx:
                fire_idx(kk + 6, (sl + 6) % NIDX)
            wait_gather(kk, b, sl)
            scale(kk, b, sl)
            fire_scatter(kk, b, sl)

        for kk in range(4):
            fire_idx(kk, kk)

        r0 = rows_bufs[0]

        def zero_rows(i, _):
            for q in range(N_FEAT // LANES):
                r0[i, pl.ds(q * LANES, LANES)] = jnp.zeros((LANES,), jnp.float32)
            return 0
        lax.fori_loop(0, CHUNK, zero_rows, 0)
        for z in range(ROWS_PER_TILE // ZCHUNK):
            pltpu.sync_copy(r0, acc_sp.at[pl.ds(s * ROWS_PER_TILE + z * ZCHUNK, ZCHUNK)])
        plsc.subcore_barrier()

        wait_idx(0, 0)
        fire_gather(0, 0, 0)
        wait_idx(1, 1)
        fire_gather(1, 1, 1)
        fire_idx(4, 4)
        fire_idx(5, 5)
        step(0, 0, 0, False, True, True)
        step(1, 1, 1, False, True, True)
        def group(i, _):
            k0 = 2 + i * 8
            for j in range(8):
                kk = k0 + j
                step(kk, (2 + j) % NBUF, (2 + j) % NIDX, True, True, True)
            return 0
        lax.fori_loop(0, (CHUNKS_PER_TILE - 16) // 8, group, 0)
        for kk in range(CHUNKS_PER_TILE - 14, CHUNKS_PER_TILE):
            step(kk, kk % NBUF, kk % NIDX,
                 True,
                 kk + 2 < CHUNKS_PER_TILE,
                 kk + 6 < CHUNKS_PER_TILE)
        for kk in range(CHUNKS_PER_TILE - 2, CHUNKS_PER_TILE):
            wait_scatter(kk, kk % NBUF, kk % NIDX)
        plsc.subcore_barrier()

        for z in range(ROWS_PER_TILE // ZCHUNK):
            rr = s * ROWS_PER_TILE + z * ZCHUNK
            pltpu.sync_copy(acc_sp.at[pl.ds(rr, ZCHUNK)], r0)
            pltpu.sync_copy(r0, out_hbm.at[c, pl.ds(rr, ZCHUNK)])

    return k(table, idx3)


BLK = 1000


def _tc_linear_kernel(a_ref, b_ref, w_ref, bias_ref, o_ref):
    x = a_ref[0] + b_ref[0]
    o_ref[...] = (
        jnp.dot(x, w_ref[...], preferred_element_type=jnp.float32) + bias_ref[...]
    )


def _tc_linear(partials, weight, bias2d):
    grid = (N_NODES // BLK,)
    return pl.pallas_call(
        _tc_linear_kernel,
        grid=grid,
        in_specs=[
            pl.BlockSpec((1, BLK, N_FEAT), lambda i: (0, i, 0)),
            pl.BlockSpec((1, BLK, N_FEAT), lambda i: (1, i, 0)),
            pl.BlockSpec((N_FEAT, N_FEAT), lambda i: (0, 0)),
            pl.BlockSpec((1, N_FEAT), lambda i: (0, 0)),
        ],
        out_specs=pl.BlockSpec((BLK, N_FEAT), lambda i: (i, 0)),
        out_shape=jax.ShapeDtypeStruct((N_NODES, N_FEAT), jnp.float32),
    )(partials, partials, weight, bias2d)


def kernel(batch_x, edge_index, edge_vals, weight, bias):
    src = edge_index[1].astype(jnp.int32)
    dst = edge_index[0].astype(jnp.int32)
    vals_i = edge_vals.astype(jnp.float32).view(jnp.int32)
    pad = E_PAD - N_EDGES
    if pad:
        src = jnp.concatenate([src, jnp.zeros((pad,), jnp.int32)])
        dst = jnp.concatenate([dst, jnp.zeros((pad,), jnp.int32)])
        vals_i = jnp.concatenate([vals_i, jnp.zeros((pad,), jnp.int32)])
    idx3 = jnp.stack(
        [src.reshape(NC * NS, CHUNKS_PER_TILE, CHUNK),
         dst.reshape(NC * NS, CHUNKS_PER_TILE, CHUNK),
         vals_i.reshape(NC * NS, CHUNKS_PER_TILE, CHUNK)],
        axis=2,
    )
    partials = _sc_segment_sum(batch_x, idx3)
    return _tc_linear(partials, weight, bias.reshape(1, N_FEAT))


# speedup vs baseline: 3.6344x; 1.0048x over previous
import functools

import jax
import jax.numpy as jnp
from jax import lax
from jax.experimental import pallas as pl
from jax.experimental.pallas import tpu as pltpu
from jax.experimental.pallas import tpu_sc as plsc

N_NODES = 10000
N_FEAT = 128
N_EDGES = 320000

NC = 2
NS = 16
LANES = 16

CHUNK = 64
CHUNKS_PER_TILE = 160
EDGES_PER_TILE = CHUNKS_PER_TILE * CHUNK
E_PAD = NC * NS * EDGES_PER_TILE

N_PAD = 10240
ROWS_PER_TILE = N_PAD // NS
ZCHUNK = 64

NBUF = 5
GPF = 2
SD = 3
NIDX = 10
IPF = 5


def _sc_segment_sum(table, idx3):
    mesh = plsc.VectorSubcoreMesh(
        core_axis_name="c", subcore_axis_name="s", num_cores=NC, num_subcores=NS
    )

    @functools.partial(
        pl.kernel,
        out_type=jax.ShapeDtypeStruct((NC, N_PAD, N_FEAT), jnp.float32),
        mesh=mesh,
        compiler_params=pltpu.CompilerParams(needs_layout_passes=False),
        scratch_types=(
            [pltpu.VMEM_SHARED((N_PAD, N_FEAT), jnp.float32)]
            + [pltpu.VMEM((CHUNK, N_FEAT), jnp.float32)] * NBUF
            + [pltpu.VMEM((3, CHUNK), jnp.int32)] * NIDX
            + [pltpu.SemaphoreType.DMA] * (NBUF + NBUF + NIDX + 1)
        ),
    )
    def k(table_hbm, idx_hbm, out_hbm, acc_sp, *bufs):
        rows_bufs = bufs[:NBUF]
        ibufs = bufs[NBUF:NBUF + NIDX]
        gsems = bufs[NBUF + NIDX:2 * NBUF + NIDX]
        ssems = bufs[2 * NBUF + NIDX:3 * NBUF + NIDX]
        isems = bufs[3 * NBUF + NIDX:3 * NBUF + 2 * NIDX]
        osem = bufs[3 * NBUF + 2 * NIDX]
        c = lax.axis_index("c")
        s = lax.axis_index("s")
        wid = c * NS + s

        def fire_idx(kk, sl):
            pltpu.async_copy(idx_hbm.at[wid, kk], ibufs[sl], isems[sl])

        def wait_idx(kk, sl):
            pltpu.make_async_copy(idx_hbm.at[wid, kk], ibufs[sl], isems[sl]).wait()

        def fire_gather(kk, b, sl):
            pltpu.async_copy(table_hbm.at[ibufs[sl].at[0]], rows_bufs[b], gsems[b])

        def wait_gather(kk, b, sl):
            pltpu.make_async_copy(
                table_hbm.at[ibufs[sl].at[0]], rows_bufs[b], gsems[b]).wait()

        def fire_scatter(kk, b, sl):
            pltpu.async_copy(rows_bufs[b], acc_sp.at[ibufs[sl].at[1]],
                             ssems[b], add=True)

        def wait_scatter(kk, b, sl):
            pltpu.make_async_copy(rows_bufs[b], acc_sp.at[ibufs[sl].at[1]],
                                  ssems[b]).wait()

        def scale(kk, b, sl):
            rows = rows_bufs[b]
            ibuf = ibufs[sl]

            @plsc.parallel_loop(0, CHUNK, unroll=4)
            def _(e):
                vv = plsc.bitcast(
                    plsc.load_gather(
                        ibuf,
                        [jnp.full((LANES,), 2, jnp.int32),
                         jnp.full((LANES,), e, jnp.int32)],
                    ),
                    jnp.float32,
                )
                for q in range(N_FEAT // LANES):
                    rows[e, pl.ds(q * LANES, LANES)] = (
                        rows[e, pl.ds(q * LANES, LANES)] * vv)

        def step(kk, b, sl, do_wait_scatter, do_fire_gather, do_fire_idx):
            if do_wait_scatter:
                wait_scatter(kk - SD, (b - SD) % NBUF, (sl - SD) % NIDX)
            if do_fire_gather:
                wait_idx(kk + GPF, (sl + GPF) % NIDX)
                fire_gather(kk + GPF, (b + GPF) % NBUF, (sl + GPF) % NIDX)
            if do_fire_idx:
                fire_idx(kk + IPF, (sl + IPF) % NIDX)
            wait_gather(kk, b, sl)
            scale(kk, b, sl)
            fire_scatter(kk, b, sl)

        for kk in range(IPF):
            fire_idx(kk, kk)

        r0 = rows_bufs[0]

        def zero_rows(i, _):
            for q in range(N_FEAT // LANES):
                r0[i, pl.ds(q * LANES, LANES)] = jnp.zeros((LANES,), jnp.float32)
            return 0
        lax.fori_loop(0, CHUNK, zero_rows, 0)
        nz = ROWS_PER_TILE // ZCHUNK
        for z in range(nz):
            pltpu.async_copy(
                r0, acc_sp.at[pl.ds(s * ROWS_PER_TILE + z * ZCHUNK, ZCHUNK)],
                ssems[z % NBUF])
        for z in range(nz):
            pltpu.make_async_copy(
                r0, acc_sp.at[pl.ds(s * ROWS_PER_TILE + z * ZCHUNK, ZCHUNK)],
                ssems[z % NBUF]).wait()
        plsc.subcore_barrier()

        for kk in range(GPF):
            wait_idx(kk, kk)
            fire_gather(kk, kk, kk)
        for kk in range(IPF):
            step(kk, kk % NBUF, kk % NIDX, kk >= SD, True, True)
        GLEN = 10
        n_groups = (CHUNKS_PER_TILE - 2 * IPF) // GLEN

        def group(i, _):
            k0 = IPF + i * GLEN
            for j in range(GLEN):
                kk = k0 + j
                step(kk, (IPF + j) % NBUF, (IPF + j) % NIDX, True, True, True)
            return 0
        lax.fori_loop(0, n_groups, group, 0)
        for kk in range(CHUNKS_PER_TILE - IPF, CHUNKS_PER_TILE):
            step(kk, kk % NBUF, kk % NIDX,
                 True,
                 kk + GPF < CHUNKS_PER_TILE,
                 kk + IPF < CHUNKS_PER_TILE)
        for kk in range(CHUNKS_PER_TILE - SD, CHUNKS_PER_TILE):
            wait_scatter(kk, kk % NBUF, kk % NIDX)
        plsc.subcore_barrier()

        rr = s * ROWS_PER_TILE
        pltpu.async_copy(acc_sp.at[pl.ds(rr, ROWS_PER_TILE)],
                         out_hbm.at[c, pl.ds(rr, ROWS_PER_TILE)], osem)
        pltpu.make_async_copy(acc_sp.at[pl.ds(rr, ROWS_PER_TILE)],
                              out_hbm.at[c, pl.ds(rr, ROWS_PER_TILE)], osem).wait()

    return k(table, idx3)


BLK = 1000


def _tc_linear_kernel(a_ref, b_ref, w_ref, bias_ref, o_ref):
    x = a_ref[0] + b_ref[0]
    o_ref[...] = (
        jnp.dot(x, w_ref[...], preferred_element_type=jnp.float32) + bias_ref[...]
    )


def _tc_linear(partials, weight, bias2d):
    grid = (N_NODES // BLK,)
    return pl.pallas_call(
        _tc_linear_kernel,
        grid=grid,
        in_specs=[
            pl.BlockSpec((1, BLK, N_FEAT), lambda i: (0, i, 0)),
            pl.BlockSpec((1, BLK, N_FEAT), lambda i: (1, i, 0)),
            pl.BlockSpec((N_FEAT, N_FEAT), lambda i: (0, 0)),
            pl.BlockSpec((1, N_FEAT), lambda i: (0, 0)),
        ],
        out_specs=pl.BlockSpec((BLK, N_FEAT), lambda i: (i, 0)),
        out_shape=jax.ShapeDtypeStruct((N_NODES, N_FEAT), jnp.float32),
    )(partials, partials, weight, bias2d)


def kernel(batch_x, edge_index, edge_vals, weight, bias):
    src = edge_index[1].astype(jnp.int32)
    dst = edge_index[0].astype(jnp.int32)
    vals_i = edge_vals.astype(jnp.float32).view(jnp.int32)
    pad = E_PAD - N_EDGES
    if pad:
        src = jnp.concatenate([src, jnp.zeros((pad,), jnp.int32)])
        dst = jnp.concatenate([dst, jnp.zeros((pad,), jnp.int32)])
        vals_i = jnp.concatenate([vals_i, jnp.zeros((pad,), jnp.int32)])
    idx3 = jnp.stack(
        [src.reshape(NC * NS, CHUNKS_PER_TILE, CHUNK),
         dst.reshape(NC * NS, CHUNKS_PER_TILE, CHUNK),
         vals_i.reshape(NC * NS, CHUNKS_PER_TILE, CHUNK)],
        axis=2,
    )
    partials = _sc_segment_sum(batch_x, idx3)
    return _tc_linear(partials, weight, bias.reshape(1, N_FEAT))
